# baseline (speedup 1.0000x reference)
import functools

import jax
import jax.numpy as jnp
from jax import lax
from jax.experimental import pallas as pl
from jax.experimental.pallas import tpu as pltpu
from jax.experimental.pallas import tpu_sc as plsc

NUM_MOD = 8
D_MODEL = 2048
NUM_TOKENS = 4 * 8192
NC, NS = 2, 16
NW = NC * NS
B_PER_W = NUM_TOKENS // NW
GROUP = 16
N_GROUPS = B_PER_W // GROUP
LAG_P = 2


def _lookup_body(idx_hbm, table_hbm, out_hbm, idx_v, table_v, table_s, sem):
    sid = lax.axis_index("s")
    wid = sid * NC + lax.axis_index("c")
    base = wid * B_PER_W
    pltpu.sync_copy(idx_hbm.at[pl.ds(base, B_PER_W)], idx_v)
    pltpu.sync_copy(table_hbm, table_v)
    pltpu.sync_copy(table_hbm, table_s.at[pl.ds(sid * NUM_MOD, NUM_MOD)])

    def quad(q, carry):
        for j, kind in enumerate(("v", "s", "h", "v")):
            off = (4 * q + j) * GROUP
            vec = idx_v[pl.ds(off, GROUP)]
            for u in range(GROUP):
                i = vec[u]
                if kind == "v":
                    src = table_v.at[pl.ds(i, 1)]
                elif kind == "s":
                    src = table_s.at[pl.ds(sid * NUM_MOD + i, 1)]
                else:
                    src = table_hbm.at[pl.ds(i, 1)]
                pltpu.async_copy(src, out_hbm.at[pl.ds(base + off + u, 1)], sem)

            @pl.when((q > 0) | (j > 0))
            def _drain():
                pltpu.make_async_copy(
                    out_hbm.at[pl.ds(base, GROUP)], out_hbm.at[pl.ds(base, GROUP)], sem
                ).wait()

        return carry

    lax.fori_loop(0, N_GROUPS // 4, quad, 0)
    pltpu.make_async_copy(
        out_hbm.at[pl.ds(base, GROUP)], out_hbm.at[pl.ds(base, GROUP)], sem
    ).wait()


_lookup = functools.partial(
    pl.kernel,
    out_type=jax.ShapeDtypeStruct((NUM_TOKENS, D_MODEL), jnp.float32),
    mesh=plsc.VectorSubcoreMesh(core_axis_name="c", subcore_axis_name="s"),
    scratch_types=[
        pltpu.VMEM((B_PER_W,), jnp.int32),
        pltpu.VMEM((NUM_MOD, D_MODEL), jnp.float32),
        pltpu.VMEM_SHARED((NS * NUM_MOD, D_MODEL), jnp.float32),
        pltpu.SemaphoreType.DMA,
    ],
)(_lookup_body)


def kernel(modality_indices, table):
    idx = modality_indices.reshape(-1).astype(jnp.int32)
    out = _lookup(idx, table)
    return out.reshape(*modality_indices.shape, table.shape[1])


# speedup vs baseline: 18.1047x; 18.1047x over previous
import functools

import jax
import jax.numpy as jnp
from jax import lax
from jax.experimental import pallas as pl
from jax.experimental.pallas import tpu as pltpu
from jax.experimental.pallas import tpu_sc as plsc

NUM_MOD = 8
D_MODEL = 2048
NUM_TOKENS = 4 * 8192
NC, NS = 2, 16
NW = NC * NS
B_PER_W = NUM_TOKENS // NW
GROUP = 16
N_GROUPS = B_PER_W // GROUP
LAG_P = 2


def _lookup_body(idx_hbm, table_hbm, out_hbm, idx_v, table_v, table_s, sem):
    sid = lax.axis_index("s")
    wid = sid * NC + lax.axis_index("c")
    base = wid * B_PER_W
    pltpu.sync_copy(idx_hbm.at[pl.ds(base, B_PER_W)], idx_v)
    pltpu.sync_copy(table_hbm, table_v)
    pltpu.sync_copy(table_hbm, table_s.at[pl.ds(sid * NUM_MOD, NUM_MOD)])

    def pair(p, carry):
        for half in range(2):
            off = (2 * p + half) * GROUP
            vec = idx_v[pl.ds(off, GROUP)]
            for u in range(GROUP):
                i = vec[u]
                if half == 0:
                    src = table_v.at[pl.ds(i, 1)]
                else:
                    src = table_s.at[pl.ds(sid * NUM_MOD + i, 1)]
                pltpu.async_copy(src, out_hbm.at[pl.ds(base + off + u, 1)], sem)

            @pl.when((p > 0) | (half > 0))
            def _drain():
                pltpu.make_async_copy(
                    out_hbm.at[pl.ds(base, GROUP)], out_hbm.at[pl.ds(base, GROUP)], sem
                ).wait()

        return carry

    lax.fori_loop(0, N_GROUPS // 2, pair, 0)
    pltpu.make_async_copy(
        out_hbm.at[pl.ds(base, GROUP)], out_hbm.at[pl.ds(base, GROUP)], sem
    ).wait()


_lookup = functools.partial(
    pl.kernel,
    out_type=jax.ShapeDtypeStruct((NUM_TOKENS, D_MODEL), jnp.float32),
    mesh=plsc.VectorSubcoreMesh(core_axis_name="c", subcore_axis_name="s"),
    scratch_types=[
        pltpu.VMEM((B_PER_W,), jnp.int32),
        pltpu.VMEM((NUM_MOD, D_MODEL), jnp.float32),
        pltpu.VMEM_SHARED((NS * NUM_MOD, D_MODEL), jnp.float32),
        pltpu.SemaphoreType.DMA,
    ],
)(_lookup_body)


def kernel(modality_indices, table):
    idx = modality_indices.reshape(-1).astype(jnp.int32)
    out = _lookup(idx, table)
    return out.reshape(*modality_indices.shape, table.shape[1])


# speedup vs baseline: 18.7187x; 1.0339x over previous
import functools

import jax
import jax.numpy as jnp
from jax import lax
from jax.experimental import pallas as pl
from jax.experimental.pallas import tpu as pltpu
from jax.experimental.pallas import tpu_sc as plsc

NUM_MOD = 8
D_MODEL = 2048
NUM_TOKENS = 4 * 8192
NC, NS = 2, 16
NW = NC * NS
B_PER_W = NUM_TOKENS // NW
GROUP = 16
N_GROUPS = B_PER_W // GROUP
LAG_P = 2


def _lookup_body(idx_hbm, table_hbm, out_hbm, idx_v, table_v, sem):
    sid = lax.axis_index("s")
    wid = sid * NC + lax.axis_index("c")
    base = wid * B_PER_W
    pltpu.sync_copy(idx_hbm.at[pl.ds(base, B_PER_W)], idx_v)
    pltpu.sync_copy(table_hbm, table_v)

    def group(g, carry):
        off = g * GROUP
        vec = idx_v[pl.ds(off, GROUP)]
        for u in range(GROUP):
            i = vec[u]
            pltpu.async_copy(
                table_v.at[pl.ds(i, 1)], out_hbm.at[pl.ds(base + off + u, 1)], sem
            )

        @pl.when(g > 0)
        def _drain():
            pltpu.make_async_copy(
                out_hbm.at[pl.ds(base, GROUP)], out_hbm.at[pl.ds(base, GROUP)], sem
            ).wait()

        return carry

    lax.fori_loop(0, N_GROUPS, group, 0)
    pltpu.make_async_copy(
        out_hbm.at[pl.ds(base, GROUP)], out_hbm.at[pl.ds(base, GROUP)], sem
    ).wait()


_lookup = functools.partial(
    pl.kernel,
    out_type=jax.ShapeDtypeStruct((NUM_TOKENS, D_MODEL), jnp.float32),
    mesh=plsc.VectorSubcoreMesh(core_axis_name="c", subcore_axis_name="s"),
    scratch_types=[
        pltpu.VMEM((B_PER_W,), jnp.int32),
        pltpu.VMEM((NUM_MOD, D_MODEL), jnp.float32),
        pltpu.SemaphoreType.DMA,
    ],
)(_lookup_body)


def kernel(modality_indices, table):
    idx = modality_indices.reshape(-1).astype(jnp.int32)
    out = _lookup(idx, table)
    return out.reshape(*modality_indices.shape, table.shape[1])
